# baseline (speedup 1.0000x reference)
import functools

import jax
import jax.numpy as jnp
from jax import lax
from jax.experimental import pallas as pl
from jax.experimental.pallas import tpu as pltpu
from jax.experimental.pallas import tpu_sc as plsc

B, S, D = 4, 8192, 1024
K = 2048
ROWS = 2048


def _fused_body(x_ref, out_ref, ss_ref):
    x = x_ref[0]
    p = None
    for c in range(8):
        xc = x[:, c * 128:(c + 1) * 128]
        sq = xc * xc
        p = sq if p is None else sq + p
    pt = jnp.transpose(p)
    a = None
    for t in range(16):
        at = pt[8 * t:8 * t + 8, :]
        a = at if a is None else at + a
    b1 = a[0:4, :] + a[4:8, :]
    b2 = b1[0:2, :] + b1[2:4, :]
    s = b2[0:1, :] + b2[1:2, :]
    i = pl.program_id(0)
    ss_ref[pl.ds(i * (ROWS // 128), ROWS // 128)] = s.reshape(
        ROWS // 128, 128)

    @pl.when(i == B * S // ROWS - 1)
    def _sort():
        _topk_from_ss(ss_ref, out_ref)


def _fused(x):
    xr = x.reshape(B * S // ROWS, ROWS, D)
    out = pl.pallas_call(
        _fused_body,
        grid=(B * S // ROWS,),
        in_specs=[pl.BlockSpec((1, ROWS, D), lambda i: (i, 0, 0))],
        out_specs=pl.BlockSpec((B, 16, 128), lambda i: (0, 0, 0)),
        out_shape=jax.ShapeDtypeStruct((B, 16, 128), jnp.int32),
        scratch_shapes=[pltpu.VMEM((B * S // 128, 128), jnp.float32)],
    )(xr)
    return out.reshape(B * K)


def _cx(k, idx, pos, j, asc):
    m = 1 << j
    am_low = (pos & m) == 0
    if j < 7:
        ax, sh = 2, m
    else:
        ax, sh = 1, m >> 7
    kp = jnp.where(am_low, jnp.roll(k, -sh, axis=ax),
                   jnp.roll(k, sh, axis=ax))
    ip = jnp.where(am_low, jnp.roll(idx, -sh, axis=ax),
                   jnp.roll(idx, sh, axis=ax))
    pw = (kp > k) | ((kp == k) & (ip < idx))
    take = (pw == am_low) != asc
    return jnp.where(take, kp, k), jnp.where(take, ip, idx)


def _first(k1, i1, k0, i0):
    return (k1 > k0) | ((k1 == k0) & (i1 < i0))


def _topk_from_ss(n_ref, out_ref):
    k = jnp.sqrt(n_ref[...].reshape(B, 64, 128))
    row = lax.broadcasted_iota(jnp.int32, (B, 64, 128), 1)
    col = lax.broadcasted_iota(jnp.int32, (B, 64, 128), 2)
    tok = row * 128 + col
    idx = tok
    for p in range(11):
        asc = ((tok >> (p + 1)) & 1) == 1
        for j in range(p, -1, -1):
            k, idx = _cx(k, idx, tok, j, asc)
    f01 = _first(k[:, 16:32], idx[:, 16:32], k[:, 0:16], idx[:, 0:16])
    f23 = _first(k[:, 48:64], idx[:, 48:64], k[:, 32:48], idx[:, 32:48])
    km = jnp.concatenate(
        [jnp.where(f01, k[:, 16:32], k[:, 0:16]),
         jnp.where(f23, k[:, 48:64], k[:, 32:48])], axis=1)
    im = jnp.concatenate(
        [jnp.where(f01, idx[:, 16:32], idx[:, 0:16]),
         jnp.where(f23, idx[:, 48:64], idx[:, 32:48])], axis=1)
    row2 = lax.broadcasted_iota(jnp.int32, (B, 32, 128), 1)
    col2 = lax.broadcasted_iota(jnp.int32, (B, 32, 128), 2)
    pos2 = (row2 % 16) * 128 + col2
    asc2 = row2 >= 16
    for j in range(10, -1, -1):
        km, im = _cx(km, im, pos2, j, asc2)
    ff = _first(km[:, 16:32], im[:, 16:32], km[:, 0:16], im[:, 0:16])
    kf = jnp.where(ff, km[:, 16:32], km[:, 0:16])
    idf = jnp.where(ff, im[:, 16:32], im[:, 0:16])
    row3 = lax.broadcasted_iota(jnp.int32, (B, 16, 128), 1)
    col3 = lax.broadcasted_iota(jnp.int32, (B, 16, 128), 2)
    pos3 = row3 * 128 + col3
    asc3 = jnp.zeros((B, 16, 128), jnp.bool_)
    for j in range(10, -1, -1):
        kf, idf = _cx(kf, idf, pos3, j, asc3)
    b = lax.broadcasted_iota(jnp.int32, (B, 16, 128), 0)
    out_ref[...] = idf + b * S


def _topk(norms):
    out = pl.pallas_call(
        _topk_body,
        in_specs=[pl.BlockSpec((B, 64, 128), lambda: (0, 0, 0))],
        out_specs=pl.BlockSpec((B, 16, 128), lambda: (0, 0, 0)),
        out_shape=jax.ShapeDtypeStruct((B, 16, 128), jnp.int32),
    )(norms.reshape(B, 64, 128))
    return out.reshape(B * K)


def _sc_gather(xf, gid):
    info = plsc.get_sparse_core_info()
    nw = info.num_cores * info.num_subcores
    rows_w = (B * K) // nw
    chunks = (56, 56, 56, 56, 32)
    mesh = plsc.VectorSubcoreMesh(core_axis_name="c", subcore_axis_name="s")

    @functools.partial(
        pl.kernel, mesh=mesh,
        out_type=jax.ShapeDtypeStruct((B * K, D), jnp.float32),
        scratch_types=[
            pltpu.VMEM((rows_w,), jnp.int32),
            pltpu.VMEM((2, 56, D), jnp.float32),
            pltpu.SemaphoreType.DMA,
            pltpu.SemaphoreType.DMA,
            pltpu.SemaphoreType.DMA,
            pltpu.SemaphoreType.DMA,
        ],
    )
    def gather_k(x_hbm, gid_hbm, out_hbm, idx_v, bufs, g0, g1, s0, s1):
        wid = lax.axis_index("s") * info.num_cores + lax.axis_index("c")
        base = wid * rows_w
        pltpu.sync_copy(gid_hbm.at[wid], idx_v)
        gsem = (g0, g1)
        ssem = (s0, s1)
        offs = []
        o = 0
        for sz in chunks:
            offs.append(o)
            o += sz
        g = [None] * len(chunks)
        s = [None] * len(chunks)

        def start_g(ch):
            b = ch & 1
            g[ch] = pltpu.async_copy(
                x_hbm.at[idx_v.at[pl.ds(offs[ch], chunks[ch])]],
                bufs.at[b].at[pl.ds(0, chunks[ch])], gsem[b])

        def start_s(ch):
            b = ch & 1
            s[ch] = pltpu.async_copy(
                bufs.at[b].at[pl.ds(0, chunks[ch])],
                out_hbm.at[pl.ds(base + offs[ch], chunks[ch])], ssem[b])

        n = len(chunks)
        start_g(0)
        for ch in range(n):
            g[ch].wait()
            start_s(ch)
            if ch + 1 < n:
                if ch >= 1:
                    s[ch - 1].wait()
                start_g(ch + 1)
        s[n - 2].wait()
        s[n - 1].wait()

    return gather_k(xf, gid.reshape(nw, rows_w))


def kernel(x):
    gid = _fused(x)
    out = _sc_gather(x.reshape(B * S, D), gid)
    return out.reshape(B, K, D)


# speedup vs baseline: 1.0243x; 1.0243x over previous
import functools

import jax
import jax.numpy as jnp
from jax import lax
from jax.experimental import pallas as pl
from jax.experimental.pallas import tpu as pltpu
from jax.experimental.pallas import tpu_sc as plsc

B, S, D = 4, 8192, 1024
K = 2048
ROWS = 2048


def _fused_body(x_ref, out_ref, ss_ref):
    x = x_ref[0]
    p = None
    for c in range(8):
        xc = x[:, c * 128:(c + 1) * 128]
        sq = xc * xc
        p = sq if p is None else sq + p
    pt = jnp.transpose(p)
    a = None
    for t in range(16):
        at = pt[8 * t:8 * t + 8, :]
        a = at if a is None else at + a
    b1 = a[0:4, :] + a[4:8, :]
    b2 = b1[0:2, :] + b1[2:4, :]
    s = b2[0:1, :] + b2[1:2, :]
    i = pl.program_id(0)
    ss_ref[pl.ds(i * (ROWS // 128), ROWS // 128)] = s.reshape(
        ROWS // 128, 128)

    @pl.when(i == B * S // ROWS - 1)
    def _sort():
        _topk_from_ss(ss_ref, out_ref)


def _fused(x):
    xr = x.reshape(B * S // ROWS, ROWS, D)
    out = pl.pallas_call(
        _fused_body,
        grid=(B * S // ROWS,),
        in_specs=[pl.BlockSpec((1, ROWS, D), lambda i: (i, 0, 0))],
        out_specs=pl.BlockSpec((B, 16, 128), lambda i: (0, 0, 0)),
        out_shape=jax.ShapeDtypeStruct((B, 16, 128), jnp.int32),
        scratch_shapes=[pltpu.VMEM((B * S // 128, 128), jnp.float32)],
    )(xr)
    return out.reshape(B * K)


def _cx(k, idx, pos, j, asc):
    m = 1 << j
    am_low = (pos & m) == 0
    if j < 7:
        ax, sh = 2, m
    else:
        ax, sh = 1, m >> 7
    kp = jnp.where(am_low, jnp.roll(k, -sh, axis=ax),
                   jnp.roll(k, sh, axis=ax))
    ip = jnp.where(am_low, jnp.roll(idx, -sh, axis=ax),
                   jnp.roll(idx, sh, axis=ax))
    pw = (kp > k) | ((kp == k) & (ip < idx))
    take = (pw == am_low) != asc
    return jnp.where(take, kp, k), jnp.where(take, ip, idx)


def _first(k1, i1, k0, i0):
    return (k1 > k0) | ((k1 == k0) & (i1 < i0))


def _topk_from_ss(n_ref, out_ref):
    k = jnp.sqrt(n_ref[...].reshape(B, 64, 128))
    row = lax.broadcasted_iota(jnp.int32, (B, 64, 128), 1)
    col = lax.broadcasted_iota(jnp.int32, (B, 64, 128), 2)
    tok = row * 128 + col
    idx = tok
    for p in range(11):
        asc = ((tok >> (p + 1)) & 1) == 1
        for j in range(p, -1, -1):
            k, idx = _cx(k, idx, tok, j, asc)
    f01 = _first(k[:, 16:32], idx[:, 16:32], k[:, 0:16], idx[:, 0:16])
    f23 = _first(k[:, 48:64], idx[:, 48:64], k[:, 32:48], idx[:, 32:48])
    km = jnp.concatenate(
        [jnp.where(f01, k[:, 16:32], k[:, 0:16]),
         jnp.where(f23, k[:, 48:64], k[:, 32:48])], axis=1)
    im = jnp.concatenate(
        [jnp.where(f01, idx[:, 16:32], idx[:, 0:16]),
         jnp.where(f23, idx[:, 48:64], idx[:, 32:48])], axis=1)
    row2 = lax.broadcasted_iota(jnp.int32, (B, 32, 128), 1)
    col2 = lax.broadcasted_iota(jnp.int32, (B, 32, 128), 2)
    pos2 = (row2 % 16) * 128 + col2
    asc2 = row2 >= 16
    for j in range(10, -1, -1):
        km, im = _cx(km, im, pos2, j, asc2)
    ff = _first(km[:, 16:32], im[:, 16:32], km[:, 0:16], im[:, 0:16])
    kf = jnp.where(ff, km[:, 16:32], km[:, 0:16])
    idf = jnp.where(ff, im[:, 16:32], im[:, 0:16])
    row3 = lax.broadcasted_iota(jnp.int32, (B, 16, 128), 1)
    col3 = lax.broadcasted_iota(jnp.int32, (B, 16, 128), 2)
    pos3 = row3 * 128 + col3
    asc3 = jnp.zeros((B, 16, 128), jnp.bool_)
    for j in range(10, -1, -1):
        kf, idf = _cx(kf, idf, pos3, j, asc3)
    b = lax.broadcasted_iota(jnp.int32, (B, 16, 128), 0)
    out_ref[...] = idf + b * S


def _topk(norms):
    out = pl.pallas_call(
        _topk_body,
        in_specs=[pl.BlockSpec((B, 64, 128), lambda: (0, 0, 0))],
        out_specs=pl.BlockSpec((B, 16, 128), lambda: (0, 0, 0)),
        out_shape=jax.ShapeDtypeStruct((B, 16, 128), jnp.int32),
    )(norms.reshape(B, 64, 128))
    return out.reshape(B * K)


def _sc_gather(xf, gid):
    info = plsc.get_sparse_core_info()
    nw = info.num_cores * info.num_subcores
    rows_w = (B * K) // nw
    chunks = (120, 120, 16)
    mesh = plsc.VectorSubcoreMesh(core_axis_name="c", subcore_axis_name="s")

    @functools.partial(
        pl.kernel, mesh=mesh,
        out_type=jax.ShapeDtypeStruct((B * K, D), jnp.float32),
        scratch_types=[
            pltpu.VMEM((rows_w,), jnp.int32),
            pltpu.VMEM((chunks[0], D), jnp.float32),
            pltpu.SemaphoreType.DMA,
        ],
    )
    def gather_k(x_hbm, gid_hbm, out_hbm, idx_v, rows_v, sem):
        wid = lax.axis_index("s") * info.num_cores + lax.axis_index("c")
        base = wid * rows_w
        pltpu.sync_copy(gid_hbm.at[wid], idx_v)
        off = 0
        for sz in chunks:
            pltpu.async_copy(x_hbm.at[idx_v.at[pl.ds(off, sz)]],
                             rows_v.at[pl.ds(0, sz)], sem).wait()
            pltpu.sync_copy(rows_v.at[pl.ds(0, sz)],
                            out_hbm.at[pl.ds(base + off, sz)])
            off += sz

    return gather_k(xf, gid.reshape(nw, rows_w))


def kernel(x):
    gid = _fused(x)
    out = _sc_gather(x.reshape(B * S, D), gid)
    return out.reshape(B, K, D)
